# baseline (speedup 1.0000x reference)
import functools

import jax
import jax.numpy as jnp
from jax import lax
from jax.experimental import pallas as pl
from jax.experimental.pallas import tpu as pltpu
from jax.experimental.pallas import tpu_sc as plsc

_N = 10000
_NP = 10240
_E = 320000
_H = 128
_NGRAPH = 16
_EPS = 1e-5

_NC = 2
_NS = 16
_EPT = _E // (_NC * _NS)
_EC = 128
_NEC = 80
_EPAD = _NC * _NS * _NEC * _EC
_DEC = 80
_NDEC = _EPT // _DEC
_RPT = _NP // _NS
_DW = 16
_NBUF = 2

_MESH = plsc.VectorSubcoreMesh(core_axis_name="c", subcore_axis_name="s",
                               num_cores=_NC, num_subcores=_NS)

_HIGH = lax.Precision.HIGHEST
_TC_PARAMS = pltpu.CompilerParams(vmem_limit_bytes=100 * 1024 * 1024)


@functools.partial(
    pl.kernel,
    out_type=jax.ShapeDtypeStruct((_NC, _NP, _DW), jnp.float32),
    mesh=_MESH,
    scratch_types=[
        pltpu.VMEM_SHARED((_NP, _DW), jnp.float32),
        pltpu.VMEM((_DEC,), jnp.int32),
        pltpu.VMEM((_DEC, _DW), jnp.float32),
        pltpu.VMEM((_RPT, _DW), jnp.float32),
    ],
)
def _deg_scatter(dst_hbm, deg_out, deg_sp, idx_v, ones_v, stage_v):
    c = lax.axis_index("c")
    s = lax.axis_index("s")
    row0 = s * _RPT

    def fill(i, carry):
        ones_v[i, :] = jnp.full((_DW,), 1.0, jnp.float32)
        return carry

    lax.fori_loop(0, _DEC, fill, 0)

    def zfill(i, carry):
        stage_v[i, :] = jnp.zeros((_DW,), jnp.float32)
        return carry

    lax.fori_loop(0, _RPT, zfill, 0)
    pltpu.sync_copy(stage_v, deg_sp.at[pl.ds(row0, _RPT)])
    plsc.subcore_barrier()

    base = (c * _NS + s) * _EPT

    def edge(j, carry):
        pltpu.sync_copy(dst_hbm.at[pl.ds(base + j * _DEC, _DEC)], idx_v)
        pltpu.sync_copy(ones_v, deg_sp.at[idx_v], add=True)
        return carry

    lax.fori_loop(0, _NDEC, edge, 0)
    plsc.subcore_barrier()
    pltpu.sync_copy(deg_sp.at[pl.ds(row0, _RPT)], stage_v)
    pltpu.sync_copy(stage_v, deg_out.at[c, pl.ds(row0, _RPT)])


@functools.partial(
    pl.kernel,
    out_type=jax.ShapeDtypeStruct((_NC, _NP, _H), jnp.float32),
    mesh=_MESH,
    scratch_types=(
        [pltpu.VMEM_SHARED((_NP, _H), jnp.float32)]
        + [pltpu.VMEM((_NEC, _EC), jnp.int32)]
        + [pltpu.VMEM((_EC,), jnp.int32)] * 4
        + [pltpu.VMEM((_EC, _H), jnp.float32)] * _NBUF
        + [pltpu.SemaphoreType.DMA] * 13
    ),
)
def _edge_scatter(hts_hbm, src_hbm, dst_hbm, acc_out, acc_sp, dst_v, *ring):
    srcs = ring[:4]
    rows = ring[4:4 + _NBUF]
    isem = ring[6:10]
    gsem = ring[10:12]
    ssem = ring[12:14]
    zsem = ring[14]
    rsem = ring[15:17]
    wsem = ring[17:19]
    c = lax.axis_index("c")
    s = lax.axis_index("s")
    w = c * _NS + s
    row0 = s * _RPT
    ebase = w * _NEC * _EC

    def i_issue(j, b):
        pltpu.async_copy(src_hbm.at[pl.ds(ebase + j * _EC, _EC)], srcs[b],
                         isem[b])

    def i_wait(j, b):
        pltpu.make_async_copy(src_hbm.at[pl.ds(ebase + j * _EC, _EC)], srcs[b],
                              isem[b]).wait()

    def g_issue(si, b):
        pltpu.async_copy(hts_hbm.at[srcs[si]], rows[b], gsem[b])

    def g_wait(si, b):
        pltpu.make_async_copy(hts_hbm.at[srcs[si]], rows[b], gsem[b]).wait()

    def s_issue(j, b):
        pltpu.async_copy(rows[b], acc_sp.at[dst_v.at[j]], ssem[b], add=True)

    def s_wait(j, b):
        pltpu.make_async_copy(rows[b], acc_sp.at[dst_v.at[j]], ssem[b]).wait()

    i_issue(0, 0)
    i_issue(1, 1)
    i_issue(2, 2)
    pltpu.sync_copy(dst_hbm.at[pl.ds(w * _NEC, _NEC)], dst_v)

    def zfill(i, carry):
        for hsub in range(_H // 16):
            rows[0][i, pl.ds(hsub * 16, 16)] = jnp.zeros((16,), jnp.float32)
        return carry

    lax.fori_loop(0, _EC, zfill, 0)
    nz = _RPT // _EC
    for k in range(nz):
        pltpu.async_copy(rows[0], acc_sp.at[pl.ds(row0 + k * _EC, _EC)], zsem)
    for k in range(nz):
        pltpu.make_async_copy(rows[0], acc_sp.at[pl.ds(row0 + k * _EC, _EC)],
                              zsem).wait()
    i_wait(0, 0)
    g_issue(0, 0)
    plsc.subcore_barrier()

    i_wait(1, 1)
    g_issue(1, 1)
    i_issue(3, 3)
    g_wait(0, 0)
    s_issue(0, 0)

    @pl.loop(1, _NEC - 3, step=4)
    def _steady(g):
        for bi in range(4):
            j = g + bi
            b = bi % 2
            sn = (bi + 2) % 4
            i_wait(j + 1, sn)
            s_wait(j - 1, b)
            g_issue(sn, b)
            i_issue(j + 3, bi)
            g_wait((bi + 1) % 4, 1 - b)
            s_issue(j, 1 - b)

    i_wait(_NEC - 2, (_NEC - 2) % 4)
    s_wait(_NEC - 4, 0)
    g_issue((_NEC - 2) % 4, 0)
    g_wait((_NEC - 3) % 4, 1)
    s_issue(_NEC - 3, 1)
    i_wait(_NEC - 1, (_NEC - 1) % 4)
    s_wait(_NEC - 3, 1)
    g_issue((_NEC - 1) % 4, 1)
    g_wait((_NEC - 2) % 4, 0)
    s_issue(_NEC - 2, 0)
    s_wait(_NEC - 2, 0)
    g_wait((_NEC - 1) % 4, 1)
    s_issue(_NEC - 1, 1)
    s_wait(_NEC - 1, 1)
    plsc.subcore_barrier()

    def d_read(k):
        pltpu.async_copy(acc_sp.at[pl.ds(row0 + k * _EC, _EC)], rows[k % 2],
                         rsem[k % 2])

    def dr_wait(k):
        pltpu.make_async_copy(acc_sp.at[pl.ds(row0 + k * _EC, _EC)],
                              rows[k % 2], rsem[k % 2]).wait()

    def d_write(k):
        pltpu.async_copy(rows[k % 2], acc_out.at[c, pl.ds(row0 + k * _EC, _EC)],
                         wsem[k % 2])

    def dw_wait(k):
        pltpu.make_async_copy(rows[k % 2],
                              acc_out.at[c, pl.ds(row0 + k * _EC, _EC)],
                              wsem[k % 2]).wait()

    d_read(0)
    d_read(1)
    for k in range(nz):
        dr_wait(k)
        d_write(k)
        if k + 2 < nz:
            dw_wait(k)
            d_read(k + 2)
    dw_wait(nz - 2)
    dw_wait(nz - 1)


def _pad_rows(a):
    return jnp.concatenate(
        [a, jnp.zeros((_NP - _N, a.shape[1]), jnp.float32)], axis=0)


def _embed_body(x_ref, wet_ref, be_ref, w0t_ref, deg_ref, h_ref, hts_ref,
                dinv_ref):
    deg = deg_ref[0, 0:_N, 0:1] + deg_ref[1, 0:_N, 0:1] + 1.0
    dinv = lax.rsqrt(deg)
    h = jnp.maximum(
        jnp.dot(x_ref[...], wet_ref[...], precision=_HIGH,
                preferred_element_type=jnp.float32) + be_ref[...], 0.0)
    h_ref[...] = h
    hts = jnp.dot(h, w0t_ref[...], precision=_HIGH,
                  preferred_element_type=jnp.float32) * dinv
    hts_ref[...] = _pad_rows(hts)
    dinv_ref[...] = dinv


_embed_tc = pl.pallas_call(
    _embed_body,
    out_shape=[
        jax.ShapeDtypeStruct((_N, _H), jnp.float32),
        jax.ShapeDtypeStruct((_NP, _H), jnp.float32),
        jax.ShapeDtypeStruct((_N, 1), jnp.float32),
    ],
    compiler_params=_TC_PARAMS,
)


def _bn_relu(acc_ref, hts_ref, h_ref, dinv_ref, b_ref, g_ref, bt_ref):
    aggp = (acc_ref[0, 0:_N, :] + acc_ref[1, 0:_N, :] + hts_ref[0:_N, :])
    agg = aggp * dinv_ref[...] + b_ref[...]
    mean = jnp.mean(agg, axis=0, keepdims=True)
    var = jnp.mean((agg - mean) ** 2, axis=0, keepdims=True)
    agg = (agg - mean) * lax.rsqrt(var + _EPS) * g_ref[...] + bt_ref[...]
    return jnp.maximum(agg, 0.0) + h_ref[...]


def _layer_body(acc_ref, hts_ref, h_ref, dinv_ref, b_ref, g_ref, bt_ref,
                wnt_ref, hout_ref, htsout_ref):
    h = _bn_relu(acc_ref, hts_ref, h_ref, dinv_ref, b_ref, g_ref, bt_ref)
    hout_ref[...] = h
    hts = jnp.dot(h, wnt_ref[...], precision=_HIGH,
                  preferred_element_type=jnp.float32) * dinv_ref[...]
    htsout_ref[...] = _pad_rows(hts)


_layer_tc = pl.pallas_call(
    _layer_body,
    out_shape=[
        jax.ShapeDtypeStruct((_N, _H), jnp.float32),
        jax.ShapeDtypeStruct((_NP, _H), jnp.float32),
    ],
    compiler_params=_TC_PARAMS,
)


def _final_body(acc_ref, hts_ref, h_ref, dinv_ref, b_ref, g_ref, bt_ref,
                batch_ref, w1t_ref, b1_ref, w2t_ref, b2_ref, out_ref):
    h = _bn_relu(acc_ref, hts_ref, h_ref, dinv_ref, b_ref, g_ref, bt_ref)
    oh = (batch_ref[...] == lax.broadcasted_iota(jnp.int32, (1, _NGRAPH), 1)
          ).astype(jnp.float32)
    sums = lax.dot_general(oh, h, (((0,), (0,)), ((), ())), precision=_HIGH,
                           preferred_element_type=jnp.float32)
    counts = lax.dot_general(oh, jnp.ones((_N, 1), jnp.float32),
                             (((0,), (0,)), ((), ())), precision=_HIGH,
                             preferred_element_type=jnp.float32)
    pooled = sums / jnp.maximum(counts, 1.0)
    z = jnp.maximum(
        jnp.dot(pooled, w1t_ref[...], precision=_HIGH,
                preferred_element_type=jnp.float32) + b1_ref[...], 0.0)
    out_ref[...] = jnp.dot(z, w2t_ref[...], precision=_HIGH,
                           preferred_element_type=jnp.float32) + b2_ref[...]


_final_tc = pl.pallas_call(
    _final_body,
    out_shape=jax.ShapeDtypeStruct((_NGRAPH, _H), jnp.float32),
    compiler_params=_TC_PARAMS,
)


def kernel(x, edge_index, batch, W_embed, b_embed, gcn_W, gcn_b, bn_gamma,
           bn_beta, head_W1, head_b1, head_W2, head_b2):
    pad = jnp.full((2, _EPAD - _E), _N, dtype=edge_index.dtype)
    eip = jnp.concatenate([edge_index, pad], axis=1)
    src = eip[0]
    dst = eip[1].reshape(_EPAD // _EC, _EC)
    deg_parts = _deg_scatter(edge_index[1])
    h, hts, dinv = _embed_tc(x, W_embed.T, b_embed.reshape(1, _H), gcn_W[0].T,
                             deg_parts)
    out = None
    for l in range(3):
        acc = _edge_scatter(hts, src, dst)
        b_l = gcn_b[l].reshape(1, _H)
        g_l = bn_gamma[l].reshape(1, _H)
        bt_l = bn_beta[l].reshape(1, _H)
        if l < 2:
            h, hts = _layer_tc(acc, hts, h, dinv, b_l, g_l, bt_l,
                               gcn_W[l + 1].T)
        else:
            out = _final_tc(acc, hts, h, dinv, b_l, g_l, bt_l,
                            batch.reshape(_N, 1), head_W1.T,
                            head_b1.reshape(1, _H), head_W2.T,
                            head_b2.reshape(1, _H))
    return out


# speedup vs baseline: 3.6756x; 3.6756x over previous
import functools

import jax
import jax.numpy as jnp
from jax import lax
from jax.experimental import pallas as pl
from jax.experimental.pallas import tpu as pltpu
from jax.experimental.pallas import tpu_sc as plsc

_N = 10000
_NP = 10240
_E = 320000
_H = 128
_NGRAPH = 16
_EPS = 1e-5

_NC = 2
_NS = 16
_EPT = _E // (_NC * _NS)
_EC = 128
_NEC = 80
_EPAD = _NC * _NS * _NEC * _EC
_DEC = 80
_NDEC = _EPT // _DEC
_RPT = _NP // _NS
_DW = 16
_NBUF = 2

_MESH = plsc.VectorSubcoreMesh(core_axis_name="c", subcore_axis_name="s",
                               num_cores=_NC, num_subcores=_NS)

_HIGH = lax.Precision.HIGHEST
_TC_PARAMS = pltpu.CompilerParams(vmem_limit_bytes=100 * 1024 * 1024)


@functools.partial(
    pl.kernel,
    out_type=jax.ShapeDtypeStruct((_NC, _NP, _DW), jnp.float32),
    mesh=_MESH,
    scratch_types=[
        pltpu.VMEM_SHARED((_NP, _DW), jnp.float32),
        pltpu.VMEM((_DEC,), jnp.int32),
        pltpu.VMEM((_DEC, _DW), jnp.float32),
        pltpu.VMEM((_RPT, _DW), jnp.float32),
    ],
)
def _deg_scatter(dst_hbm, deg_out, deg_sp, idx_v, ones_v, stage_v):
    c = lax.axis_index("c")
    s = lax.axis_index("s")
    row0 = s * _RPT

    def fill(i, carry):
        ones_v[i, :] = jnp.full((_DW,), 1.0, jnp.float32)
        return carry

    lax.fori_loop(0, _DEC, fill, 0)

    def zfill(i, carry):
        stage_v[i, :] = jnp.zeros((_DW,), jnp.float32)
        return carry

    lax.fori_loop(0, _RPT, zfill, 0)
    pltpu.sync_copy(stage_v, deg_sp.at[pl.ds(row0, _RPT)])
    plsc.subcore_barrier()

    base = (c * _NS + s) * _EPT

    def edge(j, carry):
        pltpu.sync_copy(dst_hbm.at[pl.ds(base + j * _DEC, _DEC)], idx_v)
        pltpu.sync_copy(ones_v, deg_sp.at[idx_v], add=True)
        return carry

    lax.fori_loop(0, _NDEC, edge, 0)
    plsc.subcore_barrier()
    pltpu.sync_copy(deg_sp.at[pl.ds(row0, _RPT)], stage_v)
    pltpu.sync_copy(stage_v, deg_out.at[c, pl.ds(row0, _RPT)])


@functools.partial(
    pl.kernel,
    out_type=jax.ShapeDtypeStruct((_NC, _NP, _H), jnp.float32),
    mesh=_MESH,
    scratch_types=(
        [pltpu.VMEM_SHARED((_NP, _H), jnp.float32)]
        + [pltpu.VMEM((_NEC, _EC), jnp.int32)]
        + [pltpu.VMEM((_EC,), jnp.int32)] * 4
        + [pltpu.VMEM((_EC, _H), jnp.float32)] * _NBUF
        + [pltpu.SemaphoreType.DMA] * 13
    ),
)
def _edge_scatter(hts_hbm, src_hbm, dst_hbm, acc_out, acc_sp, dst_v, *ring):
    srcs = ring[:4]
    rows = ring[4:4 + _NBUF]
    isem = ring[6:10]
    gsem = ring[10:12]
    ssem = ring[12:14]
    zsem = ring[14]
    rsem = ring[15:17]
    wsem = ring[17:19]
    c = lax.axis_index("c")
    s = lax.axis_index("s")
    w = c * _NS + s
    row0 = s * _RPT
    ebase = w * _NEC * _EC

    def i_issue(j, b):
        pltpu.async_copy(src_hbm.at[pl.ds(ebase + j * _EC, _EC)], srcs[b],
                         isem[b])

    def i_wait(j, b):
        pltpu.make_async_copy(src_hbm.at[pl.ds(ebase + j * _EC, _EC)], srcs[b],
                              isem[b]).wait()

    def g_issue(si, b):
        pltpu.async_copy(hts_hbm.at[srcs[si]], rows[b], gsem[b])

    def g_wait(si, b):
        pltpu.make_async_copy(hts_hbm.at[srcs[si]], rows[b], gsem[b]).wait()

    def s_issue(j, b):
        pltpu.async_copy(rows[b], acc_sp.at[dst_v.at[j]], ssem[b], add=True)

    def s_wait(j, b):
        pltpu.make_async_copy(rows[b], acc_sp.at[dst_v.at[j]], ssem[b]).wait()

    i_issue(0, 0)
    i_issue(1, 1)
    i_issue(2, 2)
    pltpu.sync_copy(dst_hbm.at[pl.ds(w * _NEC, _NEC)], dst_v)

    def zfill(i, carry):
        for hsub in range(_H // 16):
            rows[0][i, pl.ds(hsub * 16, 16)] = jnp.zeros((16,), jnp.float32)
        return carry

    lax.fori_loop(0, _EC, zfill, 0)
    nz = _RPT // _EC
    for k in range(nz):
        pltpu.async_copy(rows[0], acc_sp.at[pl.ds(row0 + k * _EC, _EC)], zsem)
    for k in range(nz):
        pltpu.make_async_copy(rows[0], acc_sp.at[pl.ds(row0 + k * _EC, _EC)],
                              zsem).wait()
    i_wait(0, 0)
    g_issue(0, 0)
    plsc.subcore_barrier()

    i_wait(1, 1)
    g_issue(1, 1)
    i_issue(3, 3)
    g_wait(0, 0)
    s_issue(0, 0)

    @pl.loop(1, _NEC - 3, step=4)
    def _steady(g):
        for bi in range(4):
            j = g + bi
            b = bi % 2
            sn = (bi + 2) % 4
            i_wait(j + 1, sn)
            s_wait(j - 1, b)
            g_issue(sn, b)
            i_issue(j + 3, bi)
            g_wait((bi + 1) % 4, 1 - b)
            s_issue(j, 1 - b)

    i_wait(_NEC - 2, (_NEC - 2) % 4)
    s_wait(_NEC - 4, 0)
    g_issue((_NEC - 2) % 4, 0)
    g_wait((_NEC - 3) % 4, 1)
    s_issue(_NEC - 3, 1)
    i_wait(_NEC - 1, (_NEC - 1) % 4)
    s_wait(_NEC - 3, 1)
    g_issue((_NEC - 1) % 4, 1)
    g_wait((_NEC - 2) % 4, 0)
    s_issue(_NEC - 2, 0)
    s_wait(_NEC - 2, 0)
    g_wait((_NEC - 1) % 4, 1)
    s_issue(_NEC - 1, 1)
    s_wait(_NEC - 1, 1)
    plsc.subcore_barrier()

    def d_read(k):
        pltpu.async_copy(acc_sp.at[pl.ds(row0 + k * _EC, _EC)], rows[k % 2],
                         rsem[k % 2])

    def dr_wait(k):
        pltpu.make_async_copy(acc_sp.at[pl.ds(row0 + k * _EC, _EC)],
                              rows[k % 2], rsem[k % 2]).wait()

    def d_write(k):
        pltpu.async_copy(rows[k % 2], acc_out.at[c, pl.ds(row0 + k * _EC, _EC)],
                         wsem[k % 2])

    def dw_wait(k):
        pltpu.make_async_copy(rows[k % 2],
                              acc_out.at[c, pl.ds(row0 + k * _EC, _EC)],
                              wsem[k % 2]).wait()

    d_read(0)
    d_read(1)
    for k in range(nz):
        dr_wait(k)
        d_write(k)
        if k + 2 < nz:
            dw_wait(k)
            d_read(k + 2)
    dw_wait(nz - 2)
    dw_wait(nz - 1)


def _pad_rows(a):
    return jnp.concatenate(
        [a, jnp.zeros((_NP - _N, a.shape[1]), jnp.float32)], axis=0)


def _embed_body(x_ref, wet_ref, be_ref, w0t_ref, deg_ref, h_ref, hts_ref,
                dinv_ref):
    deg = deg_ref[0, 0:_N, 0:1] + deg_ref[1, 0:_N, 0:1] + 1.0
    dinv = lax.rsqrt(deg)
    h = jnp.maximum(
        jnp.dot(x_ref[...], wet_ref[...], precision=_HIGH,
                preferred_element_type=jnp.float32) + be_ref[...], 0.0)
    h_ref[...] = h
    hts = jnp.dot(h, w0t_ref[...], precision=_HIGH,
                  preferred_element_type=jnp.float32) * dinv
    hts_ref[...] = _pad_rows(hts)
    dinv_ref[...] = dinv


_embed_tc = pl.pallas_call(
    _embed_body,
    out_shape=[
        jax.ShapeDtypeStruct((_N, _H), jnp.float32),
        jax.ShapeDtypeStruct((_NP, _H), jnp.float32),
        jax.ShapeDtypeStruct((_N, 1), jnp.float32),
    ],
    compiler_params=_TC_PARAMS,
)


def _bn_relu(acc_ref, hts_ref, h_ref, dinv_ref, b_ref, g_ref, bt_ref):
    aggp = (acc_ref[0, 0:_N, :] + acc_ref[1, 0:_N, :] + hts_ref[0:_N, :])
    agg = aggp * dinv_ref[...] + b_ref[...]
    mean = jnp.mean(agg, axis=0, keepdims=True)
    var = jnp.mean((agg - mean) ** 2, axis=0, keepdims=True)
    agg = (agg - mean) * lax.rsqrt(var + _EPS) * g_ref[...] + bt_ref[...]
    return jnp.maximum(agg, 0.0) + h_ref[...]


def _layer_body(acc_ref, hts_ref, h_ref, dinv_ref, b_ref, g_ref, bt_ref,
                wnt_ref, hout_ref, htsout_ref):
    h = _bn_relu(acc_ref, hts_ref, h_ref, dinv_ref, b_ref, g_ref, bt_ref)
    hout_ref[...] = h
    hts = jnp.dot(h, wnt_ref[...], precision=_HIGH,
                  preferred_element_type=jnp.float32) * dinv_ref[...]
    htsout_ref[...] = _pad_rows(hts)


_layer_tc = pl.pallas_call(
    _layer_body,
    out_shape=[
        jax.ShapeDtypeStruct((_N, _H), jnp.float32),
        jax.ShapeDtypeStruct((_NP, _H), jnp.float32),
    ],
    compiler_params=_TC_PARAMS,
)


def _final_body(acc_ref, hts_ref, h_ref, dinv_ref, b_ref, g_ref, bt_ref,
                batch_ref, w1t_ref, b1_ref, w2t_ref, b2_ref, out_ref):
    h = _bn_relu(acc_ref, hts_ref, h_ref, dinv_ref, b_ref, g_ref, bt_ref)
    oh = (batch_ref[...] == lax.broadcasted_iota(jnp.int32, (1, _NGRAPH), 1)
          ).astype(jnp.float32)
    sums = lax.dot_general(oh, h, (((0,), (0,)), ((), ())), precision=_HIGH,
                           preferred_element_type=jnp.float32)
    counts = lax.dot_general(oh, jnp.ones((_N, 1), jnp.float32),
                             (((0,), (0,)), ((), ())), precision=_HIGH,
                             preferred_element_type=jnp.float32)
    pooled = sums / jnp.maximum(counts, 1.0)
    z = jnp.maximum(
        jnp.dot(pooled, w1t_ref[...], precision=_HIGH,
                preferred_element_type=jnp.float32) + b1_ref[...], 0.0)
    out_ref[...] = jnp.dot(z, w2t_ref[...], precision=_HIGH,
                           preferred_element_type=jnp.float32) + b2_ref[...]


_final_tc = pl.pallas_call(
    _final_body,
    out_shape=jax.ShapeDtypeStruct((_NGRAPH, _H), jnp.float32),
    compiler_params=_TC_PARAMS,
)


def kernel(x, edge_index, batch, W_embed, b_embed, gcn_W, gcn_b, bn_gamma,
           bn_beta, head_W1, head_b1, head_W2, head_b2):
    pad_row = _N + jnp.arange(_EPAD - _E, dtype=edge_index.dtype) % (_NP - _N)
    eip = jnp.concatenate(
        [edge_index, jnp.stack([pad_row, pad_row])], axis=1)
    src = eip[0]
    dst = eip[1].reshape(_EPAD // _EC, _EC)
    deg_parts = _deg_scatter(edge_index[1])
    h, hts, dinv = _embed_tc(x, W_embed.T, b_embed.reshape(1, _H), gcn_W[0].T,
                             deg_parts)
    out = None
    for l in range(3):
        acc = _edge_scatter(hts, src, dst)
        b_l = gcn_b[l].reshape(1, _H)
        g_l = bn_gamma[l].reshape(1, _H)
        bt_l = bn_beta[l].reshape(1, _H)
        if l < 2:
            h, hts = _layer_tc(acc, hts, h, dinv, b_l, g_l, bt_l,
                               gcn_W[l + 1].T)
        else:
            out = _final_tc(acc, hts, h, dinv, b_l, g_l, bt_l,
                            batch.reshape(_N, 1), head_W1.T,
                            head_b1.reshape(1, _H), head_W2.T,
                            head_b2.reshape(1, _H))
    return out


# speedup vs baseline: 3.7737x; 1.0267x over previous
import functools

import jax
import jax.numpy as jnp
from jax import lax
from jax.experimental import pallas as pl
from jax.experimental.pallas import tpu as pltpu
from jax.experimental.pallas import tpu_sc as plsc

_N = 10000
_NP = 10240
_E = 320000
_H = 128
_NGRAPH = 16
_EPS = 1e-5

_NC = 2
_NS = 16
_EPT = _E // (_NC * _NS)
_EC = 128
_NEC = 80
_EPAD = _NC * _NS * _NEC * _EC
_DEC = 80
_NDEC = _EPT // _DEC
_RPT = _NP // _NS
_DW = 16
_NBUF = 2

_MESH = plsc.VectorSubcoreMesh(core_axis_name="c", subcore_axis_name="s",
                               num_cores=_NC, num_subcores=_NS)

_HIGH = lax.Precision.HIGHEST
_TC_PARAMS = pltpu.CompilerParams(vmem_limit_bytes=100 * 1024 * 1024)


@functools.partial(
    pl.kernel,
    out_type=jax.ShapeDtypeStruct((_NC, _NP, _DW), jnp.float32),
    mesh=_MESH,
    scratch_types=[
        pltpu.VMEM_SHARED((_NP, _DW), jnp.float32),
        pltpu.VMEM((_DEC,), jnp.int32),
        pltpu.VMEM((_DEC, _DW), jnp.float32),
        pltpu.VMEM((_RPT, _DW), jnp.float32),
    ],
)
def _deg_scatter(dst_hbm, deg_out, deg_sp, idx_v, ones_v, stage_v):
    c = lax.axis_index("c")
    s = lax.axis_index("s")
    row0 = s * _RPT

    def fill(i, carry):
        ones_v[i, :] = jnp.full((_DW,), 1.0, jnp.float32)
        return carry

    lax.fori_loop(0, _DEC, fill, 0)

    def zfill(i, carry):
        stage_v[i, :] = jnp.zeros((_DW,), jnp.float32)
        return carry

    lax.fori_loop(0, _RPT, zfill, 0)
    pltpu.sync_copy(stage_v, deg_sp.at[pl.ds(row0, _RPT)])
    plsc.subcore_barrier()

    base = (c * _NS + s) * _EPT

    def edge(j, carry):
        pltpu.sync_copy(dst_hbm.at[pl.ds(base + j * _DEC, _DEC)], idx_v)
        pltpu.sync_copy(ones_v, deg_sp.at[idx_v], add=True)
        return carry

    lax.fori_loop(0, _NDEC, edge, 0)
    plsc.subcore_barrier()
    pltpu.sync_copy(deg_sp.at[pl.ds(row0, _RPT)], stage_v)
    pltpu.sync_copy(stage_v, deg_out.at[c, pl.ds(row0, _RPT)])


@functools.partial(
    pl.kernel,
    out_type=jax.ShapeDtypeStruct((_NC, _NP, _H), jnp.float32),
    mesh=_MESH,
    scratch_types=(
        [pltpu.VMEM_SHARED((_NP, _H), jnp.float32)]
        + [pltpu.VMEM((_NEC, _EC), jnp.int32)]
        + [pltpu.VMEM((_EC,), jnp.int32)] * 4
        + [pltpu.VMEM((_EC, _H), jnp.float32)] * _NBUF
        + [pltpu.SemaphoreType.DMA] * 13
    ),
)
def _edge_scatter(hts_hbm, src_hbm, dst_hbm, acc_out, acc_sp, dst_v, *ring):
    srcs = ring[:4]
    rows = ring[4:4 + _NBUF]
    isem = ring[6:10]
    gsem = ring[10:12]
    ssem = ring[12:14]
    zsem = ring[14]
    rsem = ring[15:17]
    wsem = ring[17:19]
    c = lax.axis_index("c")
    s = lax.axis_index("s")
    w = c * _NS + s
    row0 = s * _RPT
    ebase = w * _NEC * _EC

    def i_issue(j, b):
        pltpu.async_copy(src_hbm.at[pl.ds(ebase + j * _EC, _EC)], srcs[b],
                         isem[b])

    def i_wait(j, b):
        pltpu.make_async_copy(src_hbm.at[pl.ds(ebase + j * _EC, _EC)], srcs[b],
                              isem[b]).wait()

    def g_issue(si, b):
        pltpu.async_copy(hts_hbm.at[srcs[si]], rows[b], gsem[b])

    def g_wait(si, b):
        pltpu.make_async_copy(hts_hbm.at[srcs[si]], rows[b], gsem[b]).wait()

    def s_issue(j, b):
        pltpu.async_copy(rows[b], acc_sp.at[dst_v.at[j]], ssem[b], add=True)

    def s_wait(j, b):
        pltpu.make_async_copy(rows[b], acc_sp.at[dst_v.at[j]], ssem[b]).wait()

    i_issue(0, 0)
    i_issue(1, 1)
    i_issue(2, 2)
    pltpu.sync_copy(dst_hbm.at[pl.ds(w * _NEC, _NEC)], dst_v)

    def zfill(i, carry):
        for hsub in range(_H // 16):
            rows[0][i, pl.ds(hsub * 16, 16)] = jnp.zeros((16,), jnp.float32)
        return carry

    lax.fori_loop(0, _EC, zfill, 0)
    nz = _RPT // _EC
    for k in range(nz):
        pltpu.async_copy(rows[0], acc_sp.at[pl.ds(row0 + k * _EC, _EC)], zsem)
    for k in range(nz):
        pltpu.make_async_copy(rows[0], acc_sp.at[pl.ds(row0 + k * _EC, _EC)],
                              zsem).wait()
    i_wait(0, 0)
    g_issue(0, 0)
    plsc.subcore_barrier()

    i_wait(1, 1)
    g_issue(1, 1)
    i_issue(3, 3)
    g_wait(0, 0)
    s_issue(0, 0)

    @pl.loop(1, _NEC - 3, step=4)
    def _steady(g):
        for bi in range(4):
            j = g + bi
            b = bi % 2
            sn = (bi + 2) % 4
            i_wait(j + 1, sn)
            s_wait(j - 1, b)
            g_issue(sn, b)
            i_issue(j + 3, bi)
            g_wait((bi + 1) % 4, 1 - b)
            s_issue(j, 1 - b)

    i_wait(_NEC - 2, (_NEC - 2) % 4)
    s_wait(_NEC - 4, 0)
    g_issue((_NEC - 2) % 4, 0)
    g_wait((_NEC - 3) % 4, 1)
    s_issue(_NEC - 3, 1)
    i_wait(_NEC - 1, (_NEC - 1) % 4)
    s_wait(_NEC - 3, 1)
    g_issue((_NEC - 1) % 4, 1)
    g_wait((_NEC - 2) % 4, 0)
    s_issue(_NEC - 2, 0)
    s_wait(_NEC - 2, 0)
    g_wait((_NEC - 1) % 4, 1)
    s_issue(_NEC - 1, 1)
    s_wait(_NEC - 1, 1)
    plsc.subcore_barrier()

    def d_read(k):
        pltpu.async_copy(acc_sp.at[pl.ds(row0 + k * _EC, _EC)], rows[k % 2],
                         rsem[k % 2])

    def dr_wait(k):
        pltpu.make_async_copy(acc_sp.at[pl.ds(row0 + k * _EC, _EC)],
                              rows[k % 2], rsem[k % 2]).wait()

    def d_write(k):
        pltpu.async_copy(rows[k % 2], acc_out.at[c, pl.ds(row0 + k * _EC, _EC)],
                         wsem[k % 2])

    def dw_wait(k):
        pltpu.make_async_copy(rows[k % 2],
                              acc_out.at[c, pl.ds(row0 + k * _EC, _EC)],
                              wsem[k % 2]).wait()

    d_read(0)
    d_read(1)
    for k in range(nz):
        dr_wait(k)
        d_write(k)
        if k + 2 < nz:
            dw_wait(k)
            d_read(k + 2)
    dw_wait(nz - 2)
    dw_wait(nz - 1)


def _pad_rows(a):
    return jnp.concatenate(
        [a, jnp.zeros((_NP - _N, a.shape[1]), jnp.float32)], axis=0)


def _embed_body(x_ref, wet_ref, be_ref, w0t_ref, h_ref, ht_ref):
    h = jnp.maximum(
        jnp.dot(x_ref[...], wet_ref[...], precision=_HIGH,
                preferred_element_type=jnp.float32) + be_ref[...], 0.0)
    h_ref[...] = h
    ht_ref[...] = jnp.dot(h, w0t_ref[...], precision=_HIGH,
                          preferred_element_type=jnp.float32)


_embed_tc = pl.pallas_call(
    _embed_body,
    out_shape=[
        jax.ShapeDtypeStruct((_N, _H), jnp.float32),
        jax.ShapeDtypeStruct((_N, _H), jnp.float32),
    ],
    compiler_params=_TC_PARAMS,
)


def _scale_body(ht_ref, deg_ref, hts_ref, dinv_ref):
    deg = deg_ref[0, 0:_N, 0:1] + deg_ref[1, 0:_N, 0:1] + 1.0
    dinv = lax.rsqrt(deg)
    hts_ref[...] = _pad_rows(ht_ref[...] * dinv)
    dinv_ref[...] = dinv


_scale_tc = pl.pallas_call(
    _scale_body,
    out_shape=[
        jax.ShapeDtypeStruct((_NP, _H), jnp.float32),
        jax.ShapeDtypeStruct((_N, 1), jnp.float32),
    ],
    compiler_params=_TC_PARAMS,
)


def _bn_relu(acc_ref, hts_ref, h_ref, dinv_ref, b_ref, g_ref, bt_ref):
    aggp = (acc_ref[0, 0:_N, :] + acc_ref[1, 0:_N, :] + hts_ref[0:_N, :])
    agg = aggp * dinv_ref[...] + b_ref[...]
    mean = jnp.mean(agg, axis=0, keepdims=True)
    var = jnp.mean((agg - mean) ** 2, axis=0, keepdims=True)
    agg = (agg - mean) * lax.rsqrt(var + _EPS) * g_ref[...] + bt_ref[...]
    return jnp.maximum(agg, 0.0) + h_ref[...]


def _layer_body(acc_ref, hts_ref, h_ref, dinv_ref, b_ref, g_ref, bt_ref,
                wnt_ref, hout_ref, htsout_ref):
    h = _bn_relu(acc_ref, hts_ref, h_ref, dinv_ref, b_ref, g_ref, bt_ref)
    hout_ref[...] = h
    hts = jnp.dot(h, wnt_ref[...], precision=_HIGH,
                  preferred_element_type=jnp.float32) * dinv_ref[...]
    htsout_ref[...] = _pad_rows(hts)


_layer_tc = pl.pallas_call(
    _layer_body,
    out_shape=[
        jax.ShapeDtypeStruct((_N, _H), jnp.float32),
        jax.ShapeDtypeStruct((_NP, _H), jnp.float32),
    ],
    compiler_params=_TC_PARAMS,
)


def _final_body(acc_ref, hts_ref, h_ref, dinv_ref, b_ref, g_ref, bt_ref,
                batch_ref, w1t_ref, b1_ref, w2t_ref, b2_ref, out_ref):
    h = _bn_relu(acc_ref, hts_ref, h_ref, dinv_ref, b_ref, g_ref, bt_ref)
    oh = (batch_ref[...] == lax.broadcasted_iota(jnp.int32, (1, _NGRAPH), 1)
          ).astype(jnp.float32)
    sums = lax.dot_general(oh, h, (((0,), (0,)), ((), ())), precision=_HIGH,
                           preferred_element_type=jnp.float32)
    counts = lax.dot_general(oh, jnp.ones((_N, 1), jnp.float32),
                             (((0,), (0,)), ((), ())), precision=_HIGH,
                             preferred_element_type=jnp.float32)
    pooled = sums / jnp.maximum(counts, 1.0)
    z = jnp.maximum(
        jnp.dot(pooled, w1t_ref[...], precision=_HIGH,
                preferred_element_type=jnp.float32) + b1_ref[...], 0.0)
    out_ref[...] = jnp.dot(z, w2t_ref[...], precision=_HIGH,
                           preferred_element_type=jnp.float32) + b2_ref[...]


_final_tc = pl.pallas_call(
    _final_body,
    out_shape=jax.ShapeDtypeStruct((_NGRAPH, _H), jnp.float32),
    compiler_params=_TC_PARAMS,
)


def kernel(x, edge_index, batch, W_embed, b_embed, gcn_W, gcn_b, bn_gamma,
           bn_beta, head_W1, head_b1, head_W2, head_b2):
    pad_row = _N + jnp.arange(_EPAD - _E, dtype=edge_index.dtype) % (_NP - _N)
    eip = jnp.concatenate(
        [edge_index, jnp.stack([pad_row, pad_row])], axis=1)
    src = eip[0]
    dst = eip[1].reshape(_EPAD // _EC, _EC)
    deg_parts = _deg_scatter(edge_index[1])
    h, ht0 = _embed_tc(x, W_embed.T, b_embed.reshape(1, _H), gcn_W[0].T)
    hts, dinv = _scale_tc(ht0, deg_parts)
    out = None
    for l in range(3):
        acc = _edge_scatter(hts, src, dst)
        b_l = gcn_b[l].reshape(1, _H)
        g_l = bn_gamma[l].reshape(1, _H)
        bt_l = bn_beta[l].reshape(1, _H)
        if l < 2:
            h, hts = _layer_tc(acc, hts, h, dinv, b_l, g_l, bt_l,
                               gcn_W[l + 1].T)
        else:
            out = _final_tc(acc, hts, h, dinv, b_l, g_l, bt_l,
                            batch.reshape(_N, 1), head_W1.T,
                            head_b1.reshape(1, _H), head_W2.T,
                            head_b2.reshape(1, _H))
    return out
